# baseline (speedup 1.0000x reference)
import functools

import jax
import jax.numpy as jnp
from jax import lax
from jax.experimental import pallas as pl
from jax.experimental.pallas import tpu as pltpu
from jax.experimental.pallas import tpu_sc as plsc

NUM_WORKERS = 32
GROUP_B = 16
NBUF = 2


@functools.partial(jax.jit, static_argnums=(2,))
def _gather(idx, table, b_per_w):
    b, l = idx.shape
    v, d = table.shape
    groups = b_per_w // GROUP_B

    mesh = plsc.VectorSubcoreMesh(core_axis_name="c", subcore_axis_name="s")

    @functools.partial(
        pl.kernel,
        out_type=jax.ShapeDtypeStruct((b, l, d), jnp.float32),
        mesh=mesh,
        scratch_types=[
            pltpu.VMEM((b_per_w, l), jnp.int32),
            pltpu.VMEM((NBUF, GROUP_B, l, d), jnp.float32),
            pltpu.SemaphoreType.DMA,
            pltpu.SemaphoreType.DMA,
            pltpu.SemaphoreType.DMA,
        ],
        compiler_params=pltpu.CompilerParams(use_tc_tiling_on_sc=False),
    )
    def k(idx_hbm, table_hbm, out_hbm, idx_v, rows_v, gsem, osem0, osem1):
        wid = lax.axis_index("s") * 2 + lax.axis_index("c")
        b0 = wid * b_per_w
        pltpu.sync_copy(idx_hbm.at[pl.ds(b0, b_per_w)], idx_v)
        osems = (osem0, osem1)

        def out_slice(g):
            return out_hbm.at[pl.ds(b0 + g * GROUP_B, GROUP_B)]

        def fire_one(jb, g, buf):
            pltpu.async_copy(
                table_hbm.at[idx_v.at[g * GROUP_B + jb]],
                buf.at[jb],
                gsem,
            )

        def wait_one(jb, g, buf):
            pltpu.make_async_copy(
                table_hbm.at[idx_v.at[g * GROUP_B + jb]],
                buf.at[jb],
                gsem,
            ).wait()

        def group(g, bsel, wait_prev):
            buf = rows_v.at[bsel]
            if wait_prev:
                pltpu.make_async_copy(buf, out_slice(g - NBUF), osems[bsel]).wait()
            lax.fori_loop(0, GROUP_B, lambda jb, c: (fire_one(jb, g, buf), c)[1], 0)
            lax.fori_loop(0, GROUP_B, lambda jb, c: (wait_one(jb, g, buf), c)[1], 0)
            pltpu.async_copy(buf, out_slice(g), osems[bsel])

        group(0, 0, False)
        group(1, 1, False)

        def body(i, carry):
            group(NBUF * i + 2, 0, True)
            group(NBUF * i + 3, 1, True)
            return carry

        lax.fori_loop(0, (groups - NBUF) // NBUF, body, 0)

        for bsel in range(NBUF):
            pltpu.make_async_copy(
                rows_v.at[bsel], out_slice(groups - NBUF + bsel), osems[bsel]
            ).wait()

    return k(idx, table)


def kernel(token_ids, W):
    b, l = token_ids.shape
    b_per_w = b // NUM_WORKERS
    return _gather(token_ids.astype(jnp.int32), W, b_per_w)
